# Initial kernel scaffold; baseline (speedup 1.0000x reference)
import jax
import jax.numpy as jnp
from jax.experimental import pallas as pl


def kernel(user_emb, item_emb, row_idx, col_idx, values):
    raise NotImplementedError("write your pallas kernel here")


# speedup vs baseline: 2.8900x; 2.8900x over previous
import dataclasses
import functools

import jax
import jax.numpy as jnp
from jax import lax
from jax.experimental import pallas as pl
from jax.experimental.pallas import tpu as pltpu
from jax.experimental.pallas import tpu_sc as plsc

NU = 50000
NI = 50000
NNZ = 1600000
D = 64

LANES = 128
ROWS_PER_CHUNK = 2
E = LANES * ROWS_PER_CHUNK
NROWS = NNZ // LANES
NCHUNKS = NROWS // ROWS_PER_CHUNK
NSUB = 16
MAX_ITERS = (NCHUNKS + NSUB - 1) // NSUB
HALF = NU // 2
TILE_ROWS = 1568
ACC_ROWS = TILE_ROWS * NSUB
LAST_ROWS = HALF - (NSUB - 1) * TILE_ROWS
SPREAD_MASK = 16383

_mesh = plsc.VectorSubcoreMesh(core_axis_name="c", subcore_axis_name="s")

_cp = pltpu.CompilerParams()
if "needs_layout_passes" in pltpu.CompilerParams.__dataclass_fields__:
    _cp = dataclasses.replace(_cp, needs_layout_passes=False)
if "use_tc_tiling_on_sc" in pltpu.CompilerParams.__dataclass_fields__:
    _cp = dataclasses.replace(_cp, use_tc_tiling_on_sc=False)


def _run_direction(gidx_v, didx_v, val_v, ldst_v, rows_v, acc, sem,
                   table_hbm, out_hbm, zeros_hbm, gidx_hbm, didx_hbm,
                   val_hbm, c, s):
    lo = c * HALF
    pltpu.sync_copy(zeros_hbm, acc.at[pl.ds(s * TILE_ROWS, TILE_ROWS)])
    plsc.subcore_barrier()

    @pl.loop(0, MAX_ITERS)
    def _chunk(k):
        ci = s + k * NSUB

        @pl.when(ci < NCHUNKS)
        def _():
            r0 = ci * ROWS_PER_CHUNK
            cps = [
                pltpu.async_copy(gidx_hbm.at[pl.ds(r0, ROWS_PER_CHUNK)],
                                 gidx_v, sem),
                pltpu.async_copy(didx_hbm.at[pl.ds(r0, ROWS_PER_CHUNK)],
                                 didx_v, sem),
                pltpu.async_copy(val_hbm.at[pl.ds(r0, ROWS_PER_CHUNK)],
                                 val_v, sem),
            ]
            for cp in cps:
                cp.wait()
            cps = []
            for j in range(ROWS_PER_CHUNK):
                cps.append(pltpu.async_copy(
                    table_hbm.at[gidx_v.at[j]],
                    rows_v.at[pl.ds(j * LANES, LANES)], sem))
            for cp in cps:
                cp.wait()
            for j in range(ROWS_PER_CHUNK):
                @pl.loop(0, LANES, step=16)
                def _grp(k16):
                    d16 = didx_v[j, pl.ds(k16, 16)]
                    mask = (d16 >= lo) & (d16 < lo + HALF)
                    ld16 = jnp.where(mask, d16 - lo, d16 & SPREAD_MASK)
                    ldst_v[j, pl.ds(k16, 16)] = ld16
                    v16 = val_v[j, pl.ds(k16, 16)]
                    v16 = jnp.where(mask, v16, 0.0)
                    for e in range(16):
                        sp = v16[e]
                        gr = j * LANES + k16 + e
                        for q in range(D // 16):
                            sl = pl.ds(q * 16, 16)
                            rows_v[gr, sl] = rows_v[gr, sl] * sp
            for j in range(ROWS_PER_CHUNK):
                pltpu.sync_copy(rows_v.at[pl.ds(j * LANES, LANES)],
                                acc.at[ldst_v.at[j]], add=True)

    plsc.subcore_barrier()

    @pl.when(s < NSUB - 1)
    def _():
        pltpu.sync_copy(acc.at[pl.ds(s * TILE_ROWS, TILE_ROWS)],
                        out_hbm.at[pl.ds(lo + s * TILE_ROWS, TILE_ROWS)])

    @pl.when(s == NSUB - 1)
    def _():
        pltpu.sync_copy(
            acc.at[pl.ds((NSUB - 1) * TILE_ROWS, LAST_ROWS)],
            out_hbm.at[pl.ds(lo + (NSUB - 1) * TILE_ROWS, LAST_ROWS)])

    plsc.subcore_barrier()


@functools.partial(
    pl.kernel,
    out_type=(jax.ShapeDtypeStruct((NU, D), jnp.float32),
              jax.ShapeDtypeStruct((NI, D), jnp.float32)),
    mesh=_mesh,
    scratch_types=[
        pltpu.VMEM((ROWS_PER_CHUNK, LANES), jnp.int32),
        pltpu.VMEM((ROWS_PER_CHUNK, LANES), jnp.int32),
        pltpu.VMEM((ROWS_PER_CHUNK, LANES), jnp.float32),
        pltpu.VMEM((ROWS_PER_CHUNK, LANES), jnp.int32),
        pltpu.VMEM((E, D), jnp.float32),
        pltpu.VMEM_SHARED((ACC_ROWS, D), jnp.float32),
        pltpu.SemaphoreType.DMA,
    ],
    compiler_params=_cp,
)
def _lightgcn_sc(user_hbm, item_hbm, row_hbm, col_hbm, val_hbm, zeros_hbm,
                 uout_hbm, iout_hbm,
                 gidx_v, didx_v, val_v, ldst_v, rows_v, acc, sem):
    c = lax.axis_index("c")
    s = lax.axis_index("s")
    _run_direction(gidx_v, didx_v, val_v, ldst_v, rows_v, acc, sem,
                   item_hbm, uout_hbm, zeros_hbm, col_hbm, row_hbm, val_hbm,
                   c, s)
    _run_direction(gidx_v, didx_v, val_v, ldst_v, rows_v, acc, sem,
                   user_hbm, iout_hbm, zeros_hbm, row_hbm, col_hbm, val_hbm,
                   c, s)


def kernel(user_emb, item_emb, row_idx, col_idx, values):
    row2 = row_idx.astype(jnp.int32).reshape(NROWS, LANES)
    col2 = col_idx.astype(jnp.int32).reshape(NROWS, LANES)
    val2 = values.reshape(NROWS, LANES)
    zeros = jnp.zeros((TILE_ROWS, D), jnp.float32)
    return _lightgcn_sc(user_emb, item_emb, row2, col2, val2, zeros)


# speedup vs baseline: 3.5936x; 1.2435x over previous
import dataclasses
import functools

import jax
import jax.numpy as jnp
from jax import lax
from jax.experimental import pallas as pl
from jax.experimental.pallas import tpu as pltpu
from jax.experimental.pallas import tpu_sc as plsc

NU = 50000
NI = 50000
NNZ = 1600000
D = 64

LANES = 128
NSUB = 16
NROWS_P = 12512
CHUNKS_PER_TILE = NROWS_P // NSUB
MAIN_T = CHUNKS_PER_TILE - 2
HALF = NU // 2
TILE_ROWS = 1568
ACC_ROWS = TILE_ROWS * NSUB
LAST_ROWS = HALF - (NSUB - 1) * TILE_ROWS
SPREAD_MASK = 16383

_mesh = plsc.VectorSubcoreMesh(core_axis_name="c", subcore_axis_name="s")

_cp = pltpu.CompilerParams()
if "needs_layout_passes" in pltpu.CompilerParams.__dataclass_fields__:
    _cp = dataclasses.replace(_cp, needs_layout_passes=False)
if "use_tc_tiling_on_sc" in pltpu.CompilerParams.__dataclass_fields__:
    _cp = dataclasses.replace(_cp, use_tc_tiling_on_sc=False)


def _run_direction(ebufs, ldsts, rowss, lsems, gsems, ssems, acc,
                   table_hbm, out_hbm, zeros_hbm, pack_hbm, gslot, dslot,
                   c, s):
    lo = c * HALF
    pltpu.sync_copy(zeros_hbm, acc.at[pl.ds(s * TILE_ROWS, TILE_ROWS)])
    plsc.subcore_barrier()

    def fire_load(t, p):
        r = s + t * NSUB
        return pltpu.async_copy(pack_hbm.at[pl.ds(3 * r, 3)], ebufs[p],
                                lsems[p])

    def wait_load(p):
        pltpu.make_async_copy(pack_hbm.at[pl.ds(0, 3)], ebufs[p],
                              lsems[p]).wait()

    def fire_gather(p):
        return pltpu.async_copy(table_hbm.at[ebufs[p].at[gslot]], rowss[p],
                                gsems[p])

    def wait_gather(p):
        pltpu.make_async_copy(table_hbm.at[ebufs[p].at[gslot]], rowss[p],
                              gsems[p]).wait()

    def fire_scatter(p):
        return pltpu.async_copy(rowss[p], acc.at[ldsts[p].at[0]], ssems[p],
                                add=True)

    def wait_scatter(p):
        pltpu.make_async_copy(rowss[p], acc.at[ldsts[p].at[0]],
                              ssems[p]).wait()

    def compute(p):
        ebuf, ldst, rows = ebufs[p], ldsts[p], rowss[p]

        @pl.loop(0, LANES, step=16)
        def _grp(k16):
            d16 = ebuf[dslot, pl.ds(k16, 16)]
            mask = (d16 >= lo) & (d16 < lo + HALF)
            ld16 = jnp.where(mask, d16 - lo, d16 & SPREAD_MASK)
            ldst[0, pl.ds(k16, 16)] = ld16
            v16 = plsc.bitcast(ebuf[2, pl.ds(k16, 16)], jnp.float32)
            v16 = jnp.where(mask, v16, 0.0)
            for e in range(16):
                sp = v16[e]
                for q in range(D // 16):
                    sl = pl.ds(q * 16, 16)
                    rows[k16 + e, sl] = rows[k16 + e, sl] * sp

    fire_load(0, 0)
    wait_load(0)
    fire_gather(0)
    fire_load(1, 1)

    @pl.loop(0, MAIN_T // 2)
    def _outer(k):
        for p in (0, 1):
            t = 2 * k + p
            wait_gather(p)
            wait_load(1 - p)
            if p == 0:
                @pl.when(k >= 1)
                def _():
                    wait_scatter(1 - p)
            else:
                wait_scatter(1 - p)
            fire_gather(1 - p)
            compute(p)
            fire_scatter(p)
            fire_load(t + 2, p)

    wait_gather(0)
    wait_load(1)
    wait_scatter(1)
    fire_gather(1)
    compute(0)
    fire_scatter(0)

    wait_gather(1)
    wait_scatter(0)
    compute(1)
    fire_scatter(1)
    wait_scatter(1)

    plsc.subcore_barrier()

    @pl.when(s < NSUB - 1)
    def _():
        pltpu.sync_copy(acc.at[pl.ds(s * TILE_ROWS, TILE_ROWS)],
                        out_hbm.at[pl.ds(lo + s * TILE_ROWS, TILE_ROWS)])

    @pl.when(s == NSUB - 1)
    def _():
        pltpu.sync_copy(
            acc.at[pl.ds((NSUB - 1) * TILE_ROWS, LAST_ROWS)],
            out_hbm.at[pl.ds(lo + (NSUB - 1) * TILE_ROWS, LAST_ROWS)])

    plsc.subcore_barrier()


@functools.partial(
    pl.kernel,
    out_type=(jax.ShapeDtypeStruct((NU, D), jnp.float32),
              jax.ShapeDtypeStruct((NI, D), jnp.float32)),
    mesh=_mesh,
    scratch_types=[
        pltpu.VMEM((3, LANES), jnp.int32),
        pltpu.VMEM((3, LANES), jnp.int32),
        pltpu.VMEM((1, LANES), jnp.int32),
        pltpu.VMEM((1, LANES), jnp.int32),
        pltpu.VMEM((LANES, D), jnp.float32),
        pltpu.VMEM((LANES, D), jnp.float32),
        pltpu.VMEM_SHARED((ACC_ROWS, D), jnp.float32),
        pltpu.SemaphoreType.DMA,
        pltpu.SemaphoreType.DMA,
        pltpu.SemaphoreType.DMA,
        pltpu.SemaphoreType.DMA,
        pltpu.SemaphoreType.DMA,
        pltpu.SemaphoreType.DMA,
    ],
    compiler_params=_cp,
)
def _lightgcn_sc(user_hbm, item_hbm, pack_hbm, zeros_hbm, uout_hbm, iout_hbm,
                 ebuf0, ebuf1, ldst0, ldst1, rows0, rows1, acc,
                 lsem0, lsem1, gsem0, gsem1, ssem0, ssem1):
    c = lax.axis_index("c")
    s = lax.axis_index("s")
    ebufs = (ebuf0, ebuf1)
    ldsts = (ldst0, ldst1)
    rowss = (rows0, rows1)
    lsems = (lsem0, lsem1)
    gsems = (gsem0, gsem1)
    ssems = (ssem0, ssem1)
    _run_direction(ebufs, ldsts, rowss, lsems, gsems, ssems, acc,
                   item_hbm, uout_hbm, zeros_hbm, pack_hbm, 1, 0, c, s)
    _run_direction(ebufs, ldsts, rowss, lsems, gsems, ssems, acc,
                   user_hbm, iout_hbm, zeros_hbm, pack_hbm, 0, 1, c, s)


def kernel(user_emb, item_emb, row_idx, col_idx, values):
    pad = NROWS_P * LANES - NNZ
    row_p = jnp.concatenate(
        [row_idx.astype(jnp.int32), jnp.zeros((pad,), jnp.int32)])
    col_p = jnp.concatenate(
        [col_idx.astype(jnp.int32), jnp.zeros((pad,), jnp.int32)])
    val_p = jnp.concatenate(
        [lax.bitcast_convert_type(values, jnp.int32),
         jnp.zeros((pad,), jnp.int32)])
    pack = jnp.stack([row_p.reshape(NROWS_P, LANES),
                      col_p.reshape(NROWS_P, LANES),
                      val_p.reshape(NROWS_P, LANES)], axis=1)
    pack = pack.reshape(3 * NROWS_P, LANES)
    zeros = jnp.zeros((TILE_ROWS, D), jnp.float32)
    return _lightgcn_sc(user_emb, item_emb, pack, zeros)


# speedup vs baseline: 9.2218x; 2.5661x over previous
import dataclasses
import functools

import jax
import jax.numpy as jnp
from jax import lax
from jax.experimental import pallas as pl
from jax.experimental.pallas import tpu as pltpu
from jax.experimental.pallas import tpu_sc as plsc

NU = 50000
NI = 50000
NNZ = 1600000
D = 64
DH = D // 2

LANES = 128
NSUB = 16
NROWS_P = 12512
CHUNKS_PER_TILE = NROWS_P // NSUB
MAIN_T = CHUNKS_PER_TILE - 2
TILE_ROWS = 3128
ACC_ROWS = TILE_ROWS * NSUB
LAST_ROWS = NU - (NSUB - 1) * TILE_ROWS

_mesh = plsc.VectorSubcoreMesh(core_axis_name="c", subcore_axis_name="s")

_cp = pltpu.CompilerParams()
if "needs_layout_passes" in pltpu.CompilerParams.__dataclass_fields__:
    _cp = dataclasses.replace(_cp, needs_layout_passes=False)
if "use_tc_tiling_on_sc" in pltpu.CompilerParams.__dataclass_fields__:
    _cp = dataclasses.replace(_cp, use_tc_tiling_on_sc=False)


def _run_direction(ebufs, ldsts, rowss, lsems, gsems, ssems, acc,
                   table_hbm, out_hbm, zeros_hbm, pack_hbm, gslot, dslot,
                   c, s):
    pltpu.sync_copy(zeros_hbm, acc.at[pl.ds(s * TILE_ROWS, TILE_ROWS)])
    plsc.subcore_barrier()

    def fire_load(t, p):
        r = s + t * NSUB
        return pltpu.async_copy(pack_hbm.at[pl.ds(3 * r, 3)], ebufs[p],
                                lsems[p])

    def wait_load(p):
        pltpu.make_async_copy(pack_hbm.at[pl.ds(0, 3)], ebufs[p],
                              lsems[p]).wait()

    def fire_gather(p):
        return pltpu.async_copy(table_hbm.at[ebufs[p].at[gslot]], rowss[p],
                                gsems[p])

    def wait_gather(p):
        pltpu.make_async_copy(table_hbm.at[ebufs[p].at[gslot]], rowss[p],
                              gsems[p]).wait()

    def fire_scatter(p):
        return pltpu.async_copy(rowss[p], acc.at[ldsts[p].at[0]], ssems[p],
                                add=True)

    def wait_scatter(p):
        pltpu.make_async_copy(rowss[p], acc.at[ldsts[p].at[0]],
                              ssems[p]).wait()

    def compute(p):
        ebuf, ldst, rows = ebufs[p], ldsts[p], rowss[p]

        @pl.loop(0, LANES, step=16)
        def _grp(k16):
            sl16 = pl.ds(k16, 16)
            ldst[0, sl16] = ebuf[dslot, sl16]
            v16 = plsc.bitcast(ebuf[2, sl16], jnp.float32)
            for e in range(16):
                sp = v16[e]
                for q in range(DH // 16):
                    sl = pl.ds(q * 16, 16)
                    rows[k16 + e, sl] = rows[k16 + e, sl] * sp

    fire_load(0, 0)
    wait_load(0)
    fire_gather(0)
    fire_load(1, 1)

    @pl.loop(0, MAIN_T // 2)
    def _outer(k):
        for p in (0, 1):
            t = 2 * k + p
            wait_gather(p)
            wait_load(1 - p)
            if p == 0:
                @pl.when(k >= 1)
                def _():
                    wait_scatter(1 - p)
            else:
                wait_scatter(1 - p)
            fire_gather(1 - p)
            compute(p)
            fire_scatter(p)
            fire_load(t + 2, p)

    wait_gather(0)
    wait_load(1)
    wait_scatter(1)
    fire_gather(1)
    compute(0)
    fire_scatter(0)

    wait_gather(1)
    wait_scatter(0)
    compute(1)
    fire_scatter(1)
    wait_scatter(1)

    plsc.subcore_barrier()

    @pl.when(s < NSUB - 1)
    def _():
        pltpu.sync_copy(acc.at[pl.ds(s * TILE_ROWS, TILE_ROWS)],
                        out_hbm.at[c, pl.ds(s * TILE_ROWS, TILE_ROWS)])

    @pl.when(s == NSUB - 1)
    def _():
        pltpu.sync_copy(
            acc.at[pl.ds((NSUB - 1) * TILE_ROWS, LAST_ROWS)],
            out_hbm.at[c, pl.ds((NSUB - 1) * TILE_ROWS, LAST_ROWS)])

    plsc.subcore_barrier()


@functools.partial(
    pl.kernel,
    out_type=(jax.ShapeDtypeStruct((2, NU, DH), jnp.float32),
              jax.ShapeDtypeStruct((2, NI, DH), jnp.float32)),
    mesh=_mesh,
    scratch_types=[
        pltpu.VMEM((3, LANES), jnp.int32),
        pltpu.VMEM((3, LANES), jnp.int32),
        pltpu.VMEM((1, LANES), jnp.int32),
        pltpu.VMEM((1, LANES), jnp.int32),
        pltpu.VMEM((LANES, DH), jnp.float32),
        pltpu.VMEM((LANES, DH), jnp.float32),
        pltpu.VMEM_SHARED((ACC_ROWS, DH), jnp.float32),
        pltpu.SemaphoreType.DMA,
        pltpu.SemaphoreType.DMA,
        pltpu.SemaphoreType.DMA,
        pltpu.SemaphoreType.DMA,
        pltpu.SemaphoreType.DMA,
        pltpu.SemaphoreType.DMA,
    ],
    compiler_params=_cp,
)
def _lightgcn_sc(user_hbm, item_hbm, pack_hbm, zeros_hbm, uout_hbm, iout_hbm,
                 ebuf0, ebuf1, ldst0, ldst1, rows0, rows1, acc,
                 lsem0, lsem1, gsem0, gsem1, ssem0, ssem1):
    c = lax.axis_index("c")
    s = lax.axis_index("s")
    ebufs = (ebuf0, ebuf1)
    ldsts = (ldst0, ldst1)
    rowss = (rows0, rows1)
    lsems = (lsem0, lsem1)
    gsems = (gsem0, gsem1)
    ssems = (ssem0, ssem1)
    _run_direction(ebufs, ldsts, rowss, lsems, gsems, ssems, acc,
                   item_hbm.at[c], uout_hbm, zeros_hbm, pack_hbm, 1, 0, c, s)
    _run_direction(ebufs, ldsts, rowss, lsems, gsems, ssems, acc,
                   user_hbm.at[c], iout_hbm, zeros_hbm, pack_hbm, 0, 1, c, s)


def kernel(user_emb, item_emb, row_idx, col_idx, values):
    pad = NROWS_P * LANES - NNZ
    pad_idx = (jnp.arange(pad, dtype=jnp.int32) * 16) % NU
    row_p = jnp.concatenate([row_idx.astype(jnp.int32), pad_idx])
    col_p = jnp.concatenate([col_idx.astype(jnp.int32), pad_idx])
    val_p = jnp.concatenate(
        [lax.bitcast_convert_type(values, jnp.int32),
         jnp.zeros((pad,), jnp.int32)])
    pack = jnp.stack([row_p.reshape(NROWS_P, LANES),
                      col_p.reshape(NROWS_P, LANES),
                      val_p.reshape(NROWS_P, LANES)], axis=1)
    pack = pack.reshape(3 * NROWS_P, LANES)
    user_h = jnp.stack([user_emb[:, :DH], user_emb[:, DH:]])
    item_h = jnp.stack([item_emb[:, :DH], item_emb[:, DH:]])
    zeros = jnp.zeros((TILE_ROWS, DH), jnp.float32)
    u_h, i_h = _lightgcn_sc(user_h, item_h, pack, zeros)
    user_new = jnp.concatenate([u_h[0], u_h[1]], axis=1)
    item_new = jnp.concatenate([i_h[0], i_h[1]], axis=1)
    return user_new, item_new


# speedup vs baseline: 11.5156x; 1.2487x over previous
import dataclasses
import functools

import jax
import jax.numpy as jnp
from jax import lax
from jax.experimental import pallas as pl
from jax.experimental.pallas import tpu as pltpu
from jax.experimental.pallas import tpu_sc as plsc

NU = 50000
NI = 50000
NNZ = 1600000
D = 64
DH = D // 2

LANES = 128
SUBS = 2
CHUNK = LANES * SUBS
NSUB = 16
NROWS_P = 12544
CHUNKS_PER_TILE = NROWS_P // (NSUB * SUBS)
MAIN_T = CHUNKS_PER_TILE - 2
TILE_ROWS = 3128
ACC_ROWS = TILE_ROWS * NSUB
LAST_ROWS = NU - (NSUB - 1) * TILE_ROWS

_mesh = plsc.VectorSubcoreMesh(core_axis_name="c", subcore_axis_name="s")

_cp = pltpu.CompilerParams()
if "needs_layout_passes" in pltpu.CompilerParams.__dataclass_fields__:
    _cp = dataclasses.replace(_cp, needs_layout_passes=False)
if "use_tc_tiling_on_sc" in pltpu.CompilerParams.__dataclass_fields__:
    _cp = dataclasses.replace(_cp, use_tc_tiling_on_sc=False)


def _run_direction(ebufs, ldsts, rowss, lsems, gsems, ssems, acc,
                   table_hbm, out_hbm, zeros_hbm, pack_hbm, gslot, dslot,
                   c, s):
    pltpu.sync_copy(zeros_hbm, acc.at[pl.ds(s * TILE_ROWS, TILE_ROWS)])
    plsc.subcore_barrier()

    def fire_load(t, p):
        r = SUBS * s + t * (NSUB * SUBS)
        return pltpu.async_copy(pack_hbm.at[pl.ds(3 * r, 3 * SUBS)],
                                ebufs[p], lsems[p])

    def wait_load(p):
        pltpu.make_async_copy(pack_hbm.at[pl.ds(0, 3 * SUBS)], ebufs[p],
                              lsems[p]).wait()

    def fire_gather(p):
        for j in range(SUBS):
            pltpu.async_copy(table_hbm.at[ebufs[p].at[gslot + 3 * j]],
                             rowss[p].at[pl.ds(j * LANES, LANES)], gsems[p])

    def wait_gather(p):
        for j in range(SUBS):
            pltpu.make_async_copy(
                table_hbm.at[ebufs[p].at[gslot + 3 * j]],
                rowss[p].at[pl.ds(j * LANES, LANES)], gsems[p]).wait()

    def fire_scatter(p):
        for j in range(SUBS):
            pltpu.async_copy(rowss[p].at[pl.ds(j * LANES, LANES)],
                             acc.at[ldsts[p].at[j]], ssems[p], add=True)

    def wait_scatter(p):
        for j in range(SUBS):
            pltpu.make_async_copy(rowss[p].at[pl.ds(j * LANES, LANES)],
                                  acc.at[ldsts[p].at[j]], ssems[p]).wait()

    def compute(p):
        ebuf, ldst, rows = ebufs[p], ldsts[p], rowss[p]
        for j in range(SUBS):
            @pl.loop(0, LANES, step=16)
            def _grp(k16):
                sl16 = pl.ds(k16, 16)
                ldst[j, sl16] = ebuf[dslot + 3 * j, sl16]
                v16 = plsc.bitcast(ebuf[2 + 3 * j, sl16], jnp.float32)
                for e in range(16):
                    sp = v16[e]
                    for q in range(DH // 16):
                        sl = pl.ds(q * 16, 16)
                        r = j * LANES + k16 + e
                        rows[r, sl] = rows[r, sl] * sp

    fire_load(0, 0)
    wait_load(0)
    fire_gather(0)
    fire_load(1, 1)

    @pl.loop(0, MAIN_T // 2)
    def _outer(k):
        for p in (0, 1):
            t = 2 * k + p
            wait_gather(p)
            wait_load(1 - p)
            if p == 0:
                @pl.when(k >= 1)
                def _():
                    wait_scatter(1 - p)
            else:
                wait_scatter(1 - p)
            fire_gather(1 - p)
            compute(p)
            fire_scatter(p)
            fire_load(t + 2, p)

    wait_gather(0)
    wait_load(1)
    wait_scatter(1)
    fire_gather(1)
    compute(0)
    fire_scatter(0)

    wait_gather(1)
    wait_scatter(0)
    compute(1)
    fire_scatter(1)
    wait_scatter(1)

    plsc.subcore_barrier()

    @pl.when(s < NSUB - 1)
    def _():
        pltpu.sync_copy(acc.at[pl.ds(s * TILE_ROWS, TILE_ROWS)],
                        out_hbm.at[c, pl.ds(s * TILE_ROWS, TILE_ROWS)])

    @pl.when(s == NSUB - 1)
    def _():
        pltpu.sync_copy(
            acc.at[pl.ds((NSUB - 1) * TILE_ROWS, LAST_ROWS)],
            out_hbm.at[c, pl.ds((NSUB - 1) * TILE_ROWS, LAST_ROWS)])

    plsc.subcore_barrier()


@functools.partial(
    pl.kernel,
    out_type=(jax.ShapeDtypeStruct((2, NU, DH), jnp.float32),
              jax.ShapeDtypeStruct((2, NI, DH), jnp.float32)),
    mesh=_mesh,
    scratch_types=[
        pltpu.VMEM((3 * SUBS, LANES), jnp.int32),
        pltpu.VMEM((3 * SUBS, LANES), jnp.int32),
        pltpu.VMEM((SUBS, LANES), jnp.int32),
        pltpu.VMEM((SUBS, LANES), jnp.int32),
        pltpu.VMEM((CHUNK, DH), jnp.float32),
        pltpu.VMEM((CHUNK, DH), jnp.float32),
        pltpu.VMEM_SHARED((ACC_ROWS, DH), jnp.float32),
        pltpu.SemaphoreType.DMA,
        pltpu.SemaphoreType.DMA,
        pltpu.SemaphoreType.DMA,
        pltpu.SemaphoreType.DMA,
        pltpu.SemaphoreType.DMA,
        pltpu.SemaphoreType.DMA,
    ],
    compiler_params=_cp,
)
def _lightgcn_sc(user_hbm, item_hbm, pack_hbm, zeros_hbm, uout_hbm, iout_hbm,
                 ebuf0, ebuf1, ldst0, ldst1, rows0, rows1, acc,
                 lsem0, lsem1, gsem0, gsem1, ssem0, ssem1):
    c = lax.axis_index("c")
    s = lax.axis_index("s")
    ebufs = (ebuf0, ebuf1)
    ldsts = (ldst0, ldst1)
    rowss = (rows0, rows1)
    lsems = (lsem0, lsem1)
    gsems = (gsem0, gsem1)
    ssems = (ssem0, ssem1)
    _run_direction(ebufs, ldsts, rowss, lsems, gsems, ssems, acc,
                   item_hbm.at[c], uout_hbm, zeros_hbm, pack_hbm, 1, 0, c, s)
    _run_direction(ebufs, ldsts, rowss, lsems, gsems, ssems, acc,
                   user_hbm.at[c], iout_hbm, zeros_hbm, pack_hbm, 0, 1, c, s)


def kernel(user_emb, item_emb, row_idx, col_idx, values):
    pad = NROWS_P * LANES - NNZ
    pad_idx = (jnp.arange(pad, dtype=jnp.int32) * 16) % NU
    row_p = jnp.concatenate([row_idx.astype(jnp.int32), pad_idx])
    col_p = jnp.concatenate([col_idx.astype(jnp.int32), pad_idx])
    val_p = jnp.concatenate(
        [lax.bitcast_convert_type(values, jnp.int32),
         jnp.zeros((pad,), jnp.int32)])
    pack = jnp.stack([row_p.reshape(NROWS_P, LANES),
                      col_p.reshape(NROWS_P, LANES),
                      val_p.reshape(NROWS_P, LANES)], axis=1)
    pack = pack.reshape(3 * NROWS_P, LANES)
    user_h = jnp.stack([user_emb[:, :DH], user_emb[:, DH:]])
    item_h = jnp.stack([item_emb[:, :DH], item_emb[:, DH:]])
    zeros = jnp.zeros((TILE_ROWS, DH), jnp.float32)
    u_h, i_h = _lightgcn_sc(user_h, item_h, pack, zeros)
    user_new = jnp.concatenate([u_h[0], u_h[1]], axis=1)
    item_new = jnp.concatenate([i_h[0], i_h[1]], axis=1)
    return user_new, item_new
